# baseline (speedup 1.0000x reference)
import jax
import jax.numpy as jnp
from jax import lax
from jax.experimental import pallas as pl
from jax.experimental.pallas import tpu as pltpu
from jax.experimental.pallas import tpu_sc as plsc

B, F, D = 1024, 26, 128
P = F * (F - 1) // 2
NW = 32
BPW = B // NW
NL = 16
NCH = D // NL
OUTW = BPW * P
OUTPAD = OUTW + NL


def _sc_body(x_hbm, out_hbm, in_buf, out_buf):
    wid = lax.axis_index("s") * 2 + lax.axis_index("c")
    base = wid * BPW
    pltpu.sync_copy(x_hbm.at[pl.ds(base, BPW)], in_buf)
    lanes = lax.iota(jnp.int32, NL)

    def body_b(b, carry):
        for i in range(F - 1):
            xi = [in_buf[b, i, pl.ds(k * NL, NL)] for k in range(NCH)]

            def body_j(jo, fp, xi=xi, i=i):
                j = jo + (i + 1)
                acc = xi[0] * in_buf[b, j, pl.ds(0, NL)]
                for k in range(1, NCH):
                    acc = acc + xi[k] * in_buf[b, j, pl.ds(k * NL, NL)]
                v = jnp.sum(acc)
                plsc.store_compressed(
                    out_buf.at[pl.ds(fp, NL)], jnp.full((NL,), v), mask=lanes == 0
                )
                return fp + 1

            carry = lax.fori_loop(0, F - 1 - i, body_j, carry)
        return carry

    lax.fori_loop(0, BPW, body_b, 0)
    pltpu.sync_copy(out_buf.at[pl.ds(0, OUTW)], out_hbm.at[pl.ds(wid * OUTW, OUTW)])


def kernel(x):
    mesh = plsc.VectorSubcoreMesh(core_axis_name="c", subcore_axis_name="s")
    out_flat = pl.kernel(
        _sc_body,
        mesh=mesh,
        compiler_params=pltpu.CompilerParams(
            needs_layout_passes=False, use_tc_tiling_on_sc=False
        ),
        out_type=jax.ShapeDtypeStruct((B * P,), jnp.float32),
        scratch_types=[
            pltpu.VMEM((BPW, F, D), jnp.float32),
            pltpu.VMEM((OUTPAD,), jnp.float32),
        ],
    )(x)
    return out_flat.reshape(B, P)


# speedup vs baseline: 2.1252x; 2.1252x over previous
import jax
import jax.numpy as jnp
from jax import lax
from jax.experimental import pallas as pl
from jax.experimental.pallas import tpu as pltpu
from jax.experimental.pallas import tpu_sc as plsc

B, F, D = 1024, 26, 128
P = F * (F - 1) // 2
NW = 32
BPW = B // NW
NL = 16
NCH = D // NL
OUTW = BPW * P
OUTPAD = OUTW + NL
G = 4

_BLOCKS = [(i0, min(G, F - i0)) for i0 in range(0, F, G)]


def _off(i):
    return i * (F - 1) - i * (i - 1) // 2


def _dot8(xa, xb):
    p = [xa[k] * xb[k] for k in range(NCH)]
    while len(p) > 1:
        q = [p[2 * t] + p[2 * t + 1] for t in range(len(p) // 2)]
        if len(p) % 2:
            q.append(p[-1])
        p = q
    return p[0]


def _sc_body(x_hbm, out_hbm, in_buf, out_buf):
    wid = lax.axis_index("s") * 2 + lax.axis_index("c")
    pltpu.sync_copy(x_hbm.at[pl.ds(wid * BPW, BPW)], in_buf)
    lane0 = lax.iota(jnp.int32, NL) == 0

    def store_at(pos, v):
        plsc.store_compressed(
            out_buf.at[pl.ds(pos, NL)], jnp.full((NL,), v), mask=lane0
        )

    def body_b(b, _):
        rowbase = b * P
        for i0, g in _BLOCKS:
            held = [
                [in_buf[b, i0 + t, pl.ds(k * NL, NL)] for k in range(NCH)]
                for t in range(g)
            ]
            for ta in range(g):
                for tb in range(ta + 1, g):
                    i, j = i0 + ta, i0 + tb
                    store_at(rowbase + _off(i) + (j - i - 1),
                             jnp.sum(_dot8(held[ta], held[tb])))
            jlo = i0 + g
            if jlo >= F:
                continue

            def body_j(jo, _, i0=i0, g=g, jlo=jlo, held=held, rowbase=rowbase):
                j = jo + jlo
                xj = [in_buf[b, j, pl.ds(k * NL, NL)] for k in range(NCH)]
                for t in range(g):
                    i = i0 + t
                    store_at(rowbase + _off(i) + (jlo - i - 1) + jo,
                             jnp.sum(_dot8(held[t], xj)))
                return 0

            lax.fori_loop(0, F - jlo, body_j, 0)
        return 0

    lax.fori_loop(0, BPW, body_b, 0)
    pltpu.sync_copy(out_buf.at[pl.ds(0, OUTW)], out_hbm.at[pl.ds(wid * OUTW, OUTW)])


def kernel(x):
    mesh = plsc.VectorSubcoreMesh(core_axis_name="c", subcore_axis_name="s")
    out_flat = pl.kernel(
        _sc_body,
        mesh=mesh,
        compiler_params=pltpu.CompilerParams(
            needs_layout_passes=False, use_tc_tiling_on_sc=False
        ),
        out_type=jax.ShapeDtypeStruct((B * P,), jnp.float32),
        scratch_types=[
            pltpu.VMEM((BPW, F, D), jnp.float32),
            pltpu.VMEM((OUTPAD,), jnp.float32),
        ],
    )(x)
    return out_flat.reshape(B, P)


# speedup vs baseline: 2.1696x; 1.0209x over previous
import jax
import jax.numpy as jnp
from jax import lax
from jax.experimental import pallas as pl
from jax.experimental.pallas import tpu as pltpu
from jax.experimental.pallas import tpu_sc as plsc

B, F, D = 1024, 26, 128
P = F * (F - 1) // 2
NW = 32
BPW = B // NW
NL = 16
NCH = D // NL
OUTW = BPW * P
OUTPAD = OUTW + NL
G = 4

_BLOCKS = [(i0, min(G, F - i0)) for i0 in range(0, F, G)]


def _off(i):
    return i * (F - 1) - i * (i - 1) // 2


def _dot8(xa, xb):
    p = [xa[k] * xb[k] for k in range(NCH)]
    while len(p) > 1:
        q = [p[2 * t] + p[2 * t + 1] for t in range(len(p) // 2)]
        if len(p) % 2:
            q.append(p[-1])
        p = q
    return p[0]


def _sc_body(x_hbm, out_hbm, in_buf, out_buf):
    wid = lax.axis_index("s") * 2 + lax.axis_index("c")
    pltpu.sync_copy(x_hbm.at[pl.ds(wid * BPW, BPW)], in_buf)
    lane0 = lax.iota(jnp.int32, NL) == 0

    def store_at(pos, v):
        plsc.store_compressed(
            out_buf.at[pl.ds(pos, NL)], jnp.full((NL,), v), mask=lane0
        )

    def body_b(b, _):
        rowbase = b * P
        for i0, g in _BLOCKS:
            held = [
                [in_buf[b, i0 + t, pl.ds(k * NL, NL)] for k in range(NCH)]
                for t in range(g)
            ]
            for ta in range(g):
                for tb in range(ta + 1, g):
                    i, j = i0 + ta, i0 + tb
                    store_at(rowbase + _off(i) + (j - i - 1),
                             jnp.sum(_dot8(held[ta], held[tb])))
            jlo = i0 + g
            if jlo >= F:
                continue

            def body_j(jo, _, i0=i0, g=g, jlo=jlo, held=held, rowbase=rowbase):
                j = jo + jlo
                xj = [in_buf[b, j, pl.ds(k * NL, NL)] for k in range(NCH)]
                for t in range(g):
                    i = i0 + t
                    store_at(rowbase + _off(i) + (jlo - i - 1) + jo,
                             jnp.sum(_dot8(held[t], xj)))
                return 0

            lax.fori_loop(0, F - jlo, body_j, 0, unroll=2)
        return 0

    lax.fori_loop(0, BPW, body_b, 0)
    pltpu.sync_copy(out_buf.at[pl.ds(0, OUTW)], out_hbm.at[pl.ds(wid * OUTW, OUTW)])


def kernel(x):
    mesh = plsc.VectorSubcoreMesh(core_axis_name="c", subcore_axis_name="s")
    out_flat = pl.kernel(
        _sc_body,
        mesh=mesh,
        compiler_params=pltpu.CompilerParams(
            needs_layout_passes=False, use_tc_tiling_on_sc=False
        ),
        out_type=jax.ShapeDtypeStruct((B * P,), jnp.float32),
        scratch_types=[
            pltpu.VMEM((BPW, F, D), jnp.float32),
            pltpu.VMEM((OUTPAD,), jnp.float32),
        ],
    )(x)
    return out_flat.reshape(B, P)
